# baseline (speedup 1.0000x reference)
import functools

import jax
import jax.numpy as jnp
from jax import lax
from jax.experimental import pallas as pl
from jax.experimental.pallas import tpu as pltpu
from jax.experimental.pallas import tpu_sc as plsc

N_NODES = 10000
N_EDGES = 320000
IN_DIM = 128
HID = 64
OUT_DIM = 128

NC = 2
NS = 16
NW = NC * NS
NBINS = 128
BINROWS = 79
N_PAD = NBINS * BINROWS
CAP = 4096
E_PART = NBINS * CAP
PCH = 2560
G = 16

_SC_PARAMS = pltpu.CompilerParams(
    use_tc_tiling_on_sc=False, needs_layout_passes=False)

_mesh = plsc.VectorSubcoreMesh(
    core_axis_name="c", subcore_axis_name="s", num_cores=NC, num_subcores=NS)


def _wid():
  return lax.axis_index("s") * NC + lax.axis_index("c")


_WCAP = 12288


def _partition_body(src_hbm, dst_hbm, psrc_hbm, pdstl_hbm, cnt_hbm,
                    src_c, dst_c, bufs, cntv, ssrc, sdst):
  wid = _wid()
  iota = lax.iota(jnp.int32, 16)

  w_lo = wid * 4 * BINROWS
  w_hi = w_lo + 4 * BINROWS

  def chunk_body(t, wp):
    pltpu.sync_copy(src_hbm.at[pl.ds(t * PCH, PCH)], src_c)
    pltpu.sync_copy(dst_hbm.at[pl.ds(t * PCH, PCH)], dst_c)

    def vec_body(v, wp2):
      sv = src_c[pl.ds(v * 16, 16)]
      dv = dst_c[pl.ds(v * 16, 16)]
      m = jnp.logical_and(dv >= w_lo, dv < w_hi)
      cs = plsc.cumsum(m.astype(jnp.int32))
      pos = wp2 + cs - 1
      pos = jnp.minimum(jnp.maximum(pos, 0), _WCAP - 1)
      plsc.store_scatter(ssrc, [pos], sv, mask=m)
      plsc.store_scatter(sdst, [pos], dv - w_lo, mask=m)
      return wp2 + cs[15]

    return lax.fori_loop(0, PCH // 16, vec_body, wp)

  nw = lax.fori_loop(0, N_EDGES // PCH, chunk_body, jnp.int32(0))

  def split_body(v, wps2):
    base = v * 16
    sv = ssrc[pl.ds(base, 16)]
    dv = sdst[pl.ds(base, 16)]
    valid = (iota + base) < nw
    binv = dv // BINROWS
    bl = dv - binv * BINROWS
    new = []
    for j in range(4):
      m = jnp.logical_and(binv == j, valid)
      cs = plsc.cumsum(m.astype(jnp.int32))
      pos = wps2[j] + cs - 1
      pos = jnp.minimum(jnp.maximum(pos, 0), CAP - 1)
      plsc.store_scatter(bufs[2 * j], [pos], sv, mask=m)
      plsc.store_scatter(bufs[2 * j + 1], [pos], bl, mask=m)
      new.append(wps2[j] + cs[15])
    return tuple(new)

  nv = (nw + 15) // 16
  wps = lax.fori_loop(0, nv, split_body,
                      (jnp.int32(0), jnp.int32(0), jnp.int32(0), jnp.int32(0)))

  lane = iota
  cvec = jnp.zeros((16,), jnp.int32)
  for j in range(4):
    cvec = jnp.where(lane == j, wps[j], cvec)
  cntv[pl.ds(0, 16)] = cvec
  for j in range(4):
    bj = wid * 4 + j
    pltpu.sync_copy(bufs[2 * j], psrc_hbm.at[pl.ds(bj * CAP, CAP)])
    pltpu.sync_copy(bufs[2 * j + 1], pdstl_hbm.at[pl.ds(bj * CAP, CAP)])
  pltpu.sync_copy(cntv, cnt_hbm.at[wid])


def _make_partition():
  scratch = [pltpu.VMEM((PCH,), jnp.int32), pltpu.VMEM((PCH,), jnp.int32)]
  scratch += [pltpu.VMEM((CAP,), jnp.int32) for _ in range(8)]
  scratch += [pltpu.VMEM((16,), jnp.int32)]
  scratch += [pltpu.VMEM((_WCAP,), jnp.int32), pltpu.VMEM((_WCAP,), jnp.int32)]

  def body(src_hbm, dst_hbm, psrc_hbm, pdstl_hbm, cnt_hbm,
           src_c, dst_c, b0, b1, b2, b3, b4, b5, b6, b7, cntv, ssrc, sdst):
    _partition_body(src_hbm, dst_hbm, psrc_hbm, pdstl_hbm, cnt_hbm,
                    src_c, dst_c, (b0, b1, b2, b3, b4, b5, b6, b7), cntv,
                    ssrc, sdst)

  return pl.kernel(
      body,
      out_type=(jax.ShapeDtypeStruct((E_PART,), jnp.int32),
                jax.ShapeDtypeStruct((E_PART,), jnp.int32),
                jax.ShapeDtypeStruct((NW, 16), jnp.int32)),
      mesh=_mesh,
      compiler_params=_SC_PARAMS,
      scratch_types=scratch,
      name="gat_edge_partition")


_partition = _make_partition()


_DENPAD = ((BINROWS * 8 + 15) // 16 + 1) * 16


def _make_edge_kernel(H, D):
  HD = H * D
  HD1 = HD + 1
  SLABSZ = BINROWS * HD1
  VSZ = ((SLABSZ + 15) // 16) * 16
  ELSL = (G - 1) * HD1 + 1
  SLSL = (BINROWS - 1) * HD1 + 1

  def body(feat2d, mb_hbm, psrc, pdstl, cnt_hbm, attn_hbm,
           v_hbm, den_hbm,
           slab, v_loc, den_l, mb_l, src_l, dstl_l, el0, el1, el_pad,
           idx0, idx1, attn_l, cnt_r, sem0, sem1):
    wid = _wid()
    iota = lax.iota(jnp.int32, 16)
    zf = jnp.zeros((16,), jnp.float32)
    pltpu.sync_copy(cnt_hbm.at[wid], cnt_r)
    for h in range(H):
      pltpu.sync_copy(attn_hbm.at[h], attn_l.at[pl.ds(h * D, D)])

    zr_r = iota // 8
    zr_c = iota % 8

    def bin_body(j, _):
      b = wid * 4 + j
      lo = b * BINROWS
      cnt = plsc.load_gather(cnt_r, [jnp.full((16,), j)])[0]

      pltpu.sync_copy(feat2d.at[pl.ds(lo, BINROWS)],
                      slab.at[:, pl.ds(0, HD)])
      pltpu.sync_copy(mb_hbm.at[pl.ds(lo, BINROWS)], mb_l)
      pltpu.sync_copy(psrc.at[pl.ds(b * CAP, CAP)], src_l)
      pltpu.sync_copy(pdstl.at[pl.ds(b * CAP, CAP)], dstl_l)

      def zero_v(r, _):
        for k in range(HD // 16):
          v_loc[r, pl.ds(k * 16, 16)] = zf
        return 0
      lax.fori_loop(0, BINROWS, zero_v, 0)

      def zero_d(i, _):
        plsc.store_scatter(den_l, [i * 2 + zr_r, zr_c], zf)
        return 0
      lax.fori_loop(0, BINROWS // 2 + 1, zero_d, 0)

      ng = (cnt + (G - 1)) // G

      def issue(g, idxb, elb, semb):
        base = g * G
        sv = src_l[pl.ds(base, 16)]
        mk = (iota + base) < cnt
        idxb[pl.ds(0, 16)] = jnp.where(mk, sv, 0)
        pltpu.make_async_copy(feat2d.at[idxb], elb, semb).start()

      @pl.when(ng > 0)
      def _():
        issue(0, idx0, el0, sem0)

      def repack(elb):
        def rp(r2, _):
          r = r2 * 2
          for kb in range(0, HD // 16, 8):
            vs = [elb[r, pl.ds((kb + i) * 16, 16)] for i in range(8)]
            ws = [elb[r + 1, pl.ds((kb + i) * 16, 16)] for i in range(8)]
            for i in range(8):
              el_pad[r, pl.ds((kb + i) * 16, 16)] = vs[i]
              el_pad[r + 1, pl.ds((kb + i) * 16, 16)] = ws[i]
          return 0
        lax.fori_loop(0, G // 2, rp, 0)

      def group_body(g, _):
        even = (g % 2) == 0
        more = (g + 1) < ng

        @pl.when(jnp.logical_and(more, even))
        def _():
          issue(g + 1, idx1, el1, sem1)

        @pl.when(jnp.logical_and(more, jnp.logical_not(even)))
        def _():
          issue(g + 1, idx0, el0, sem0)

        @pl.when(even)
        def _():
          pltpu.make_async_copy(feat2d.at[idx0], el0, sem0).wait()
          repack(el0)

        @pl.when(jnp.logical_not(even))
        def _():
          pltpu.make_async_copy(feat2d.at[idx1], el1, sem1).wait()
          repack(el1)

        base = g * G
        mk = (iota + base) < cnt
        dl = dstl_l[pl.ds(base, 16)]
        dls = jnp.where(mk, dl, 0)

        def head_body(h, _):
          hD = h * D

          def blk_body(dd, accs):
            c0 = hD + dd * 16
            a_vec = attn_l[pl.ds(c0, 16)]
            acc0, acc1, ci = accs
            for kb in range(0, 16, 4):
              cis = [ci + (kb + i) for i in range(4)]
              es = [plsc.load_gather(el_pad, [iota, cis[i]])
                    for i in range(4)]
              rs = [plsc.load_gather(slab, [dls, cis[i]]) for i in range(4)]
              for i in range(4):
                w = es[i] + rs[i]
                t = jnp.maximum(w, 0.2 * w)
                if i % 2 == 0:
                  acc0 = acc0 + t * a_vec[kb + i]
                else:
                  acc1 = acc1 + t * a_vec[kb + i]
            return (acc0, acc1, ci + 16)

          ci0 = jnp.full((16,), hD)
          acc0, acc1, _ci = lax.fori_loop(0, D // 16, blk_body,
                                          (zf, zf, ci0))
          acc = acc0 + acc1
          hv = jnp.full((16,), h)
          mbg = plsc.load_gather(mb_l, [dls, hv])
          p = jnp.exp(acc - mbg)
          p = jnp.where(mk, p, 0.0)
          plsc.addupdate_scatter(den_l, [dls, hv], p)

          def acc_blk(dd, ci):
            for kb in range(0, 16, 4):
              cis = [ci + (kb + i) for i in range(4)]
              es = [plsc.load_gather(el_pad, [iota, cis[i]])
                    for i in range(4)]
              vs = [es[i] * p for i in range(4)]
              for i in range(4):
                plsc.addupdate_scatter(v_loc, [dls, cis[i]], vs[i])
            return ci + 16

          lax.fori_loop(0, D // 16, acc_blk, ci0)
          return 0

        lax.fori_loop(0, H, head_body, 0)
        return 0

      lax.fori_loop(0, ng, group_body, 0)

      pltpu.sync_copy(v_loc.at[:, pl.ds(0, HD)], v_hbm.at[pl.ds(lo, BINROWS)])
      pltpu.sync_copy(den_l.at[pl.ds(0, BINROWS)],
                      den_hbm.at[pl.ds(lo, BINROWS)])
      return 0

    lax.fori_loop(0, 4, bin_body, 0)

  scratch = [
      pltpu.VMEM((BINROWS, HD1), jnp.float32),
      pltpu.VMEM((BINROWS, HD1), jnp.float32),
      pltpu.VMEM((BINROWS + 1, 8), jnp.float32),
      pltpu.VMEM((BINROWS, 8), jnp.float32),
      pltpu.VMEM((CAP,), jnp.int32),
      pltpu.VMEM((CAP,), jnp.int32),
      pltpu.VMEM((G, HD), jnp.float32),
      pltpu.VMEM((G, HD), jnp.float32),
      pltpu.VMEM((G, HD1), jnp.float32),
      pltpu.VMEM((G,), jnp.int32),
      pltpu.VMEM((G,), jnp.int32),
      pltpu.VMEM((HD,), jnp.float32),
      pltpu.VMEM((16,), jnp.int32),
      pltpu.SemaphoreType.DMA,
      pltpu.SemaphoreType.DMA,
  ]
  return pl.kernel(
      body,
      out_type=(jax.ShapeDtypeStruct((N_PAD, HD), jnp.float32),
                jax.ShapeDtypeStruct((N_PAD, 8), jnp.float32)),
      mesh=_mesh,
      compiler_params=_SC_PARAMS,
      scratch_types=scratch,
      name=f"gat_edge_h{H}d{D}")


_edge_8_64 = _make_edge_kernel(8, HID)
_edge_1_128 = _make_edge_kernel(1, OUT_DIM)


_BR = 632
_GRID = N_PAD // _BR


def _mm0_body(x_ref, w_ref, a_ref, feat_ref, s_ref):
  f = jnp.dot(x_ref[...], w_ref[...], preferred_element_type=jnp.float32)
  feat_ref[...] = f
  s_ref[...] = jnp.dot(jnp.abs(f), a_ref[...],
                       preferred_element_type=jnp.float32)


def _mm0(x, w, a_abs):
  F = x.shape[1]
  HD = w.shape[1]
  return pl.pallas_call(
      _mm0_body,
      grid=(_GRID,),
      in_specs=[pl.BlockSpec((_BR, F), lambda i: (i, 0)),
                pl.BlockSpec((F, HD), lambda i: (0, 0)),
                pl.BlockSpec((HD, 8), lambda i: (0, 0))],
      out_specs=[pl.BlockSpec((_BR, HD), lambda i: (i, 0)),
                 pl.BlockSpec((_BR, 8), lambda i: (i, 0))],
      out_shape=[jax.ShapeDtypeStruct((N_PAD, HD), jnp.float32),
                 jax.ShapeDtypeStruct((N_PAD, 8), jnp.float32)],
  )(x, w, a_abs)


def _mmn_body(v_ref, den_ref, b_ref, exp_ref, w_ref, a_ref, feat_ref, s_ref):
  den_e = jnp.dot(den_ref[...], exp_ref[...],
                  preferred_element_type=jnp.float32)
  h = v_ref[...] / (den_e + 1e-9) + b_ref[...]
  h = jnp.where(h > 0, h, jnp.exp(jnp.minimum(h, 0.0)) - 1.0)
  f = jnp.dot(h, w_ref[...], preferred_element_type=jnp.float32)
  feat_ref[...] = f
  s_ref[...] = jnp.dot(jnp.abs(f), a_ref[...],
                       preferred_element_type=jnp.float32)


def _mm_norm(v, den, brow, exp8, w, a_abs):
  HDp = v.shape[1]
  HD = w.shape[1]
  return pl.pallas_call(
      _mmn_body,
      grid=(_GRID,),
      in_specs=[pl.BlockSpec((_BR, HDp), lambda i: (i, 0)),
                pl.BlockSpec((_BR, 8), lambda i: (i, 0)),
                pl.BlockSpec((1, HDp), lambda i: (0, 0)),
                pl.BlockSpec((8, HDp), lambda i: (0, 0)),
                pl.BlockSpec((HDp, HD), lambda i: (0, 0)),
                pl.BlockSpec((HD, 8), lambda i: (0, 0))],
      out_specs=[pl.BlockSpec((_BR, HD), lambda i: (i, 0)),
                 pl.BlockSpec((_BR, 8), lambda i: (i, 0))],
      out_shape=[jax.ShapeDtypeStruct((N_PAD, HD), jnp.float32),
                 jax.ShapeDtypeStruct((N_PAD, 8), jnp.float32)],
  )(v, den, brow, exp8, w, a_abs)


def _mb_body(s_ref, mb_ref):
  s = s_ref[...]
  mb_ref[...] = s + jnp.max(s)


def _mb(s):
  return pl.pallas_call(
      _mb_body,
      out_shape=jax.ShapeDtypeStruct((N_PAD, 8), jnp.float32),
  )(s)


def _readout_body(v_ref, den_ref, b_ref, o_ref):
  logits = v_ref[...] / (den_ref[...][:, 0:1] + 1e-9) + b_ref[...]
  rows = lax.broadcasted_iota(jnp.int32, (N_PAD, OUT_DIM), 0)
  logits = jnp.where(rows < N_NODES, logits, -1e30)
  o_ref[...] = jnp.max(logits, axis=0, keepdims=True)


def _readout(v, den, brow):
  return pl.pallas_call(
      _readout_body,
      out_shape=jax.ShapeDtypeStruct((1, OUT_DIM), jnp.float32),
  )(v, den, brow)


def _head_expander(h_heads, hd):
  d = hd // h_heads
  heads = jnp.repeat(jnp.arange(h_heads), d)
  return jax.nn.one_hot(heads, 8, dtype=jnp.float32)


def kernel(x, edge_index, W0, attn0, b0, W1, attn1, b1, W2, attn2, b2):
  x = x.astype(jnp.float32)
  src = edge_index[0].astype(jnp.int32)
  dst = edge_index[1].astype(jnp.int32)

  x_pad = jnp.pad(x, ((0, N_PAD - N_NODES), (0, 0)))

  psrc, pdstl, counts = _partition(src, dst)

  def prep_a(attn, hd):
    oh = _head_expander(attn.shape[0], hd)
    return oh * jnp.abs(attn).reshape(-1, 1)

  a0 = prep_a(attn0, 8 * HID)
  a1 = prep_a(attn1, 8 * HID)
  a2 = prep_a(attn2, OUT_DIM)
  exp8_512 = _head_expander(8, 8 * HID).T

  feat0, s0 = _mm0(x_pad, W0, a0)
  v0, den0 = _edge_8_64(feat0, _mb(s0), psrc, pdstl, counts, attn0)

  feat1, s1 = _mm_norm(v0, den0, b0.reshape(1, -1), exp8_512, W1, a1)
  v1, den1 = _edge_8_64(feat1, _mb(s1), psrc, pdstl, counts, attn1)

  feat2, s2 = _mm_norm(v1, den1, b1.reshape(1, -1), exp8_512, W2, a2)
  v2, den2 = _edge_1_128(feat2, _mb(s2), psrc, pdstl, counts, attn2)

  return _readout(v2, den2, b2.reshape(1, -1))


# speedup vs baseline: 1.0340x; 1.0340x over previous
import functools

import jax
import jax.numpy as jnp
from jax import lax
from jax.experimental import pallas as pl
from jax.experimental.pallas import tpu as pltpu
from jax.experimental.pallas import tpu_sc as plsc

N_NODES = 10000
N_EDGES = 320000
IN_DIM = 128
HID = 64
OUT_DIM = 128

NC = 2
NS = 16
NW = NC * NS
NBINS = 128
BINROWS = 79
N_PAD = NBINS * BINROWS
CAP = 4096
E_PART = NBINS * CAP
PCH = 2560
G = 16

_SC_PARAMS = pltpu.CompilerParams(
    use_tc_tiling_on_sc=False, needs_layout_passes=False)

_mesh = plsc.VectorSubcoreMesh(
    core_axis_name="c", subcore_axis_name="s", num_cores=NC, num_subcores=NS)


def _wid():
  return lax.axis_index("s") * NC + lax.axis_index("c")


_WCAP = 12288


def _partition_body(src_hbm, dst_hbm, psrc_hbm, pdstl_hbm, cnt_hbm,
                    src_c, dst_c, bufs, cntv, ssrc, sdst):
  wid = _wid()
  iota = lax.iota(jnp.int32, 16)

  w_lo = wid * 4 * BINROWS
  w_hi = w_lo + 4 * BINROWS

  def chunk_body(t, wp):
    pltpu.sync_copy(src_hbm.at[pl.ds(t * PCH, PCH)], src_c)
    pltpu.sync_copy(dst_hbm.at[pl.ds(t * PCH, PCH)], dst_c)

    def vec_body(v, wp2):
      sv = src_c[pl.ds(v * 16, 16)]
      dv = dst_c[pl.ds(v * 16, 16)]
      m = jnp.logical_and(dv >= w_lo, dv < w_hi)
      cs = plsc.cumsum(m.astype(jnp.int32))
      pos = wp2 + cs - 1
      pos = jnp.minimum(jnp.maximum(pos, 0), _WCAP - 1)
      plsc.store_scatter(ssrc, [pos], sv, mask=m)
      plsc.store_scatter(sdst, [pos], dv - w_lo, mask=m)
      return wp2 + cs[15]

    return lax.fori_loop(0, PCH // 16, vec_body, wp)

  nw = lax.fori_loop(0, N_EDGES // PCH, chunk_body, jnp.int32(0))

  def split_body(v, wps2):
    base = v * 16
    sv = ssrc[pl.ds(base, 16)]
    dv = sdst[pl.ds(base, 16)]
    valid = (iota + base) < nw
    binv = dv // BINROWS
    bl = dv - binv * BINROWS
    new = []
    for j in range(4):
      m = jnp.logical_and(binv == j, valid)
      cs = plsc.cumsum(m.astype(jnp.int32))
      pos = wps2[j] + cs - 1
      pos = jnp.minimum(jnp.maximum(pos, 0), CAP - 1)
      plsc.store_scatter(bufs[2 * j], [pos], sv, mask=m)
      plsc.store_scatter(bufs[2 * j + 1], [pos], bl, mask=m)
      new.append(wps2[j] + cs[15])
    return tuple(new)

  nv = (nw + 15) // 16
  wps = lax.fori_loop(0, nv, split_body,
                      (jnp.int32(0), jnp.int32(0), jnp.int32(0), jnp.int32(0)))

  lane = iota
  cvec = jnp.zeros((16,), jnp.int32)
  for j in range(4):
    cvec = jnp.where(lane == j, wps[j], cvec)
  cntv[pl.ds(0, 16)] = cvec
  for j in range(4):
    bj = wid * 4 + j
    pltpu.sync_copy(bufs[2 * j], psrc_hbm.at[pl.ds(bj * CAP, CAP)])
    pltpu.sync_copy(bufs[2 * j + 1], pdstl_hbm.at[pl.ds(bj * CAP, CAP)])
  pltpu.sync_copy(cntv, cnt_hbm.at[wid])


def _make_partition():
  scratch = [pltpu.VMEM((PCH,), jnp.int32), pltpu.VMEM((PCH,), jnp.int32)]
  scratch += [pltpu.VMEM((CAP,), jnp.int32) for _ in range(8)]
  scratch += [pltpu.VMEM((16,), jnp.int32)]
  scratch += [pltpu.VMEM((_WCAP,), jnp.int32), pltpu.VMEM((_WCAP,), jnp.int32)]

  def body(src_hbm, dst_hbm, psrc_hbm, pdstl_hbm, cnt_hbm,
           src_c, dst_c, b0, b1, b2, b3, b4, b5, b6, b7, cntv, ssrc, sdst):
    _partition_body(src_hbm, dst_hbm, psrc_hbm, pdstl_hbm, cnt_hbm,
                    src_c, dst_c, (b0, b1, b2, b3, b4, b5, b6, b7), cntv,
                    ssrc, sdst)

  return pl.kernel(
      body,
      out_type=(jax.ShapeDtypeStruct((E_PART,), jnp.int32),
                jax.ShapeDtypeStruct((E_PART,), jnp.int32),
                jax.ShapeDtypeStruct((NW, 16), jnp.int32)),
      mesh=_mesh,
      compiler_params=_SC_PARAMS,
      scratch_types=scratch,
      name="gat_edge_partition")


_partition = _make_partition()


_DENPAD = ((BINROWS * 8 + 15) // 16 + 1) * 16


def _make_edge_kernel(H, D):
  HD = H * D
  HD1 = HD + 1
  SLABSZ = BINROWS * HD1
  VSZ = ((SLABSZ + 15) // 16) * 16
  ELSL = (G - 1) * HD1 + 1
  SLSL = (BINROWS - 1) * HD1 + 1

  def body(feat2d, mb_hbm, psrc, pdstl, cnt_hbm, attn_hbm,
           v_hbm, den_hbm,
           slab, v_loc, den_l, mb_l, src_l, dstl_l, el0, el1, el_pad,
           idx0, idx1, attn_l, cnt_r, sem0, sem1):
    wid = _wid()
    iota = lax.iota(jnp.int32, 16)
    zf = jnp.zeros((16,), jnp.float32)
    pltpu.sync_copy(cnt_hbm.at[wid], cnt_r)
    for h in range(H):
      pltpu.sync_copy(attn_hbm.at[h], attn_l.at[pl.ds(h * D, D)])

    zr_r = iota // 8
    zr_c = iota % 8

    def bin_body(j, _):
      b = wid * 4 + j
      lo = b * BINROWS
      cnt = plsc.load_gather(cnt_r, [jnp.full((16,), j)])[0]

      pltpu.sync_copy(feat2d.at[pl.ds(lo, BINROWS)],
                      slab.at[:, pl.ds(0, HD)])
      pltpu.sync_copy(mb_hbm.at[pl.ds(lo, BINROWS)], mb_l)
      pltpu.sync_copy(psrc.at[pl.ds(b * CAP, CAP)], src_l)
      pltpu.sync_copy(pdstl.at[pl.ds(b * CAP, CAP)], dstl_l)

      def zero_v(r, _):
        for k in range(HD // 16):
          v_loc[r, pl.ds(k * 16, 16)] = zf
        return 0
      lax.fori_loop(0, BINROWS, zero_v, 0)

      def zero_d(i, _):
        plsc.store_scatter(den_l, [i * 2 + zr_r, zr_c], zf)
        return 0
      lax.fori_loop(0, BINROWS // 2 + 1, zero_d, 0)

      ng = (cnt + (G - 1)) // G

      def issue(g, idxb, elb, semb):
        base = g * G
        sv = src_l[pl.ds(base, 16)]
        mk = (iota + base) < cnt
        idxb[pl.ds(0, 16)] = jnp.where(mk, sv, 0)
        pltpu.make_async_copy(feat2d.at[idxb], elb, semb).start()

      @pl.when(ng > 0)
      def _():
        issue(0, idx0, el0, sem0)

      def repack(elb):
        def rp(r, _):
          for kb in range(0, HD // 16, 8):
            vs = [elb[r, pl.ds((kb + i) * 16, 16)] for i in range(8)]
            for i in range(8):
              el_pad[r, pl.ds((kb + i) * 16, 16)] = vs[i]
          return 0
        lax.fori_loop(0, G, rp, 0)

      def group_body(g, _):
        even = (g % 2) == 0
        more = (g + 1) < ng

        @pl.when(jnp.logical_and(more, even))
        def _():
          issue(g + 1, idx1, el1, sem1)

        @pl.when(jnp.logical_and(more, jnp.logical_not(even)))
        def _():
          issue(g + 1, idx0, el0, sem0)

        @pl.when(even)
        def _():
          pltpu.make_async_copy(feat2d.at[idx0], el0, sem0).wait()
          repack(el0)

        @pl.when(jnp.logical_not(even))
        def _():
          pltpu.make_async_copy(feat2d.at[idx1], el1, sem1).wait()
          repack(el1)

        base = g * G
        mk = (iota + base) < cnt
        dl = dstl_l[pl.ds(base, 16)]
        dls = jnp.where(mk, dl, 0)

        def head_body(h, _):
          hD = h * D

          def blk_body(dd, accs):
            c0 = hD + dd * 16
            a_vec = attn_l[pl.ds(c0, 16)]
            acc0, acc1, ci = accs
            for kb in range(0, 16, 8):
              cis = [ci + (kb + i) for i in range(8)]
              es = [plsc.load_gather(el_pad, [iota, cis[i]])
                    for i in range(8)]
              rs = [plsc.load_gather(slab, [dls, cis[i]]) for i in range(8)]
              for i in range(8):
                w = es[i] + rs[i]
                t = jnp.maximum(w, 0.2 * w)
                if i % 2 == 0:
                  acc0 = acc0 + t * a_vec[kb + i]
                else:
                  acc1 = acc1 + t * a_vec[kb + i]
            return (acc0, acc1, ci + 16)

          ci0 = jnp.full((16,), hD)
          acc0, acc1, _ci = lax.fori_loop(0, D // 16, blk_body,
                                          (zf, zf, ci0))
          acc = acc0 + acc1
          hv = jnp.full((16,), h)
          mbg = plsc.load_gather(mb_l, [dls, hv])
          p = jnp.exp(acc - mbg)
          p = jnp.where(mk, p, 0.0)
          plsc.addupdate_scatter(den_l, [dls, hv], p)

          def acc_blk(dd, ci):
            for kb in range(0, 16, 8):
              cis = [ci + (kb + i) for i in range(8)]
              es = [plsc.load_gather(el_pad, [iota, cis[i]])
                    for i in range(8)]
              vs = [es[i] * p for i in range(8)]
              for i in range(8):
                plsc.addupdate_scatter(v_loc, [dls, cis[i]], vs[i])
            return ci + 16

          lax.fori_loop(0, D // 16, acc_blk, ci0)
          return 0

        lax.fori_loop(0, H, head_body, 0)
        return 0

      lax.fori_loop(0, ng, group_body, 0)

      pltpu.sync_copy(v_loc.at[:, pl.ds(0, HD)], v_hbm.at[pl.ds(lo, BINROWS)])
      pltpu.sync_copy(den_l.at[pl.ds(0, BINROWS)],
                      den_hbm.at[pl.ds(lo, BINROWS)])
      return 0

    lax.fori_loop(0, 4, bin_body, 0)

  scratch = [
      pltpu.VMEM((BINROWS, HD1), jnp.float32),
      pltpu.VMEM((BINROWS, HD1), jnp.float32),
      pltpu.VMEM((BINROWS + 1, 8), jnp.float32),
      pltpu.VMEM((BINROWS, 8), jnp.float32),
      pltpu.VMEM((CAP,), jnp.int32),
      pltpu.VMEM((CAP,), jnp.int32),
      pltpu.VMEM((G, HD), jnp.float32),
      pltpu.VMEM((G, HD), jnp.float32),
      pltpu.VMEM((G, HD1), jnp.float32),
      pltpu.VMEM((G,), jnp.int32),
      pltpu.VMEM((G,), jnp.int32),
      pltpu.VMEM((HD,), jnp.float32),
      pltpu.VMEM((16,), jnp.int32),
      pltpu.SemaphoreType.DMA,
      pltpu.SemaphoreType.DMA,
  ]
  return pl.kernel(
      body,
      out_type=(jax.ShapeDtypeStruct((N_PAD, HD), jnp.float32),
                jax.ShapeDtypeStruct((N_PAD, 8), jnp.float32)),
      mesh=_mesh,
      compiler_params=_SC_PARAMS,
      scratch_types=scratch,
      name=f"gat_edge_h{H}d{D}")


_edge_8_64 = _make_edge_kernel(8, HID)
_edge_1_128 = _make_edge_kernel(1, OUT_DIM)


_BR = 632
_GRID = N_PAD // _BR


def _mm0_body(x_ref, w_ref, a_ref, feat_ref, s_ref):
  f = jnp.dot(x_ref[...], w_ref[...], preferred_element_type=jnp.float32)
  feat_ref[...] = f
  s_ref[...] = jnp.dot(jnp.abs(f), a_ref[...],
                       preferred_element_type=jnp.float32)


def _mm0(x, w, a_abs):
  F = x.shape[1]
  HD = w.shape[1]
  return pl.pallas_call(
      _mm0_body,
      grid=(_GRID,),
      in_specs=[pl.BlockSpec((_BR, F), lambda i: (i, 0)),
                pl.BlockSpec((F, HD), lambda i: (0, 0)),
                pl.BlockSpec((HD, 8), lambda i: (0, 0))],
      out_specs=[pl.BlockSpec((_BR, HD), lambda i: (i, 0)),
                 pl.BlockSpec((_BR, 8), lambda i: (i, 0))],
      out_shape=[jax.ShapeDtypeStruct((N_PAD, HD), jnp.float32),
                 jax.ShapeDtypeStruct((N_PAD, 8), jnp.float32)],
  )(x, w, a_abs)


def _mmn_body(v_ref, den_ref, b_ref, exp_ref, w_ref, a_ref, feat_ref, s_ref):
  den_e = jnp.dot(den_ref[...], exp_ref[...],
                  preferred_element_type=jnp.float32)
  h = v_ref[...] / (den_e + 1e-9) + b_ref[...]
  h = jnp.where(h > 0, h, jnp.exp(jnp.minimum(h, 0.0)) - 1.0)
  f = jnp.dot(h, w_ref[...], preferred_element_type=jnp.float32)
  feat_ref[...] = f
  s_ref[...] = jnp.dot(jnp.abs(f), a_ref[...],
                       preferred_element_type=jnp.float32)


def _mm_norm(v, den, brow, exp8, w, a_abs):
  HDp = v.shape[1]
  HD = w.shape[1]
  return pl.pallas_call(
      _mmn_body,
      grid=(_GRID,),
      in_specs=[pl.BlockSpec((_BR, HDp), lambda i: (i, 0)),
                pl.BlockSpec((_BR, 8), lambda i: (i, 0)),
                pl.BlockSpec((1, HDp), lambda i: (0, 0)),
                pl.BlockSpec((8, HDp), lambda i: (0, 0)),
                pl.BlockSpec((HDp, HD), lambda i: (0, 0)),
                pl.BlockSpec((HD, 8), lambda i: (0, 0))],
      out_specs=[pl.BlockSpec((_BR, HD), lambda i: (i, 0)),
                 pl.BlockSpec((_BR, 8), lambda i: (i, 0))],
      out_shape=[jax.ShapeDtypeStruct((N_PAD, HD), jnp.float32),
                 jax.ShapeDtypeStruct((N_PAD, 8), jnp.float32)],
  )(v, den, brow, exp8, w, a_abs)


def _mb_body(s_ref, mb_ref):
  s = s_ref[...]
  mb_ref[...] = s + jnp.max(s)


def _mb(s):
  return pl.pallas_call(
      _mb_body,
      out_shape=jax.ShapeDtypeStruct((N_PAD, 8), jnp.float32),
  )(s)


def _readout_body(v_ref, den_ref, b_ref, o_ref):
  logits = v_ref[...] / (den_ref[...][:, 0:1] + 1e-9) + b_ref[...]
  rows = lax.broadcasted_iota(jnp.int32, (N_PAD, OUT_DIM), 0)
  logits = jnp.where(rows < N_NODES, logits, -1e30)
  o_ref[...] = jnp.max(logits, axis=0, keepdims=True)


def _readout(v, den, brow):
  return pl.pallas_call(
      _readout_body,
      out_shape=jax.ShapeDtypeStruct((1, OUT_DIM), jnp.float32),
  )(v, den, brow)


def _head_expander(h_heads, hd):
  d = hd // h_heads
  heads = jnp.repeat(jnp.arange(h_heads), d)
  return jax.nn.one_hot(heads, 8, dtype=jnp.float32)


def kernel(x, edge_index, W0, attn0, b0, W1, attn1, b1, W2, attn2, b2):
  x = x.astype(jnp.float32)
  src = edge_index[0].astype(jnp.int32)
  dst = edge_index[1].astype(jnp.int32)

  x_pad = jnp.pad(x, ((0, N_PAD - N_NODES), (0, 0)))

  psrc, pdstl, counts = _partition(src, dst)

  def prep_a(attn, hd):
    oh = _head_expander(attn.shape[0], hd)
    return oh * jnp.abs(attn).reshape(-1, 1)

  a0 = prep_a(attn0, 8 * HID)
  a1 = prep_a(attn1, 8 * HID)
  a2 = prep_a(attn2, OUT_DIM)
  exp8_512 = _head_expander(8, 8 * HID).T

  feat0, s0 = _mm0(x_pad, W0, a0)
  v0, den0 = _edge_8_64(feat0, _mb(s0), psrc, pdstl, counts, attn0)

  feat1, s1 = _mm_norm(v0, den0, b0.reshape(1, -1), exp8_512, W1, a1)
  v1, den1 = _edge_8_64(feat1, _mb(s1), psrc, pdstl, counts, attn1)

  feat2, s2 = _mm_norm(v1, den1, b1.reshape(1, -1), exp8_512, W2, a2)
  v2, den2 = _edge_1_128(feat2, _mb(s2), psrc, pdstl, counts, attn2)

  return _readout(v2, den2, b2.reshape(1, -1))
